# baseline (speedup 1.0000x reference)
import functools

import numpy as np
import jax
import jax.numpy as jnp
from jax import lax
from jax.experimental import pallas as pl
from jax.experimental.pallas import tpu as pltpu
from jax.experimental.pallas import tpu_sc as plsc

_VOCAB = 100000
_HID = 128
_PAD = 0
_MAXPOS = 16384
_BIAS = 1
_EPS = 1e-12

_NW = 32
_BLK = 128


def _sinusoidal_table(num_embeddings, embedding_dim, padding_idx):
    half_dim = embedding_dim // 2
    emb = np.log(10000.0) / (half_dim - 1)
    emb = np.exp(np.arange(half_dim, dtype=np.float64) * -emb)
    emb = np.arange(num_embeddings, dtype=np.float64)[:, None] * emb[None, :]
    table = np.concatenate([np.sin(emb), np.cos(emb)], axis=1)
    if embedding_dim % 2 == 1:
        table = np.concatenate([table, np.zeros((num_embeddings, 1))], axis=1)
    if padding_idx is not None:
        table[padding_idx, :] = 0.0
    return np.asarray(table, dtype=np.float32)


_POS_TABLE = _sinusoidal_table(_MAXPOS, _HID, _PAD)


def _sc_body(S, CHUNK, ROW_W,
             ids_hbm, wtab_hbm, ptab_hbm, lnw_hbm, lnb_hbm, out_hbm,
             row_ids, pos_idx, wbuf0, wbuf1, pbuf0, pbuf1, obuf0, obuf1,
             lnw_v, lnb_v,
             semw0, semw1, semp0, semp1, semo0, semo1):
    wid = lax.axis_index("c") * 16 + lax.axis_index("s")
    b = wid // ROW_W
    c = wid % ROW_W
    NBLK = CHUNK // _BLK

    wbufs = [wbuf0, wbuf1]
    pbufs = [pbuf0, pbuf1]
    obufs = [obuf0, obuf1]
    semws = [semw0, semw1]
    semps = [semp0, semp1]
    semos = [semo0, semo1]

    pltpu.sync_copy(ids_hbm.at[pl.ds(b * S, S)], row_ids)
    pltpu.sync_copy(lnw_hbm, lnw_v)
    pltpu.sync_copy(lnb_hbm, lnb_v)

    def issue_w(blk):
        s = blk % 2
        return pltpu.async_copy(
            wtab_hbm.at[row_ids.at[pl.ds(c * CHUNK + blk * _BLK, _BLK)]],
            wbufs[s], semws[s])

    def issue_p(blk):
        s = blk % 2
        return pltpu.async_copy(
            ptab_hbm.at[pos_idx.at[pl.ds(blk * _BLK, _BLK)]],
            pbufs[s], semps[s])

    cw = [None] * NBLK
    cp = [None] * NBLK
    co = [None] * NBLK
    cw[0] = issue_w(0)
    cw[1] = issue_w(1)

    nv = c * (CHUNK // 16)

    def pc_body(j, acc):
        v = row_ids[pl.ds(j * 16, 16)]
        return acc + jnp.minimum(v, 1)

    lane15 = jnp.full((16,), 15, jnp.int32)

    def splat_last(x):
        return x.at[lane15].get(mode="promise_in_bounds")

    acc = lax.fori_loop(0, nv, pc_body, jnp.zeros((16,), jnp.int32))
    cnt0 = splat_last(plsc.cumsum(acc))

    def pos_body(j, cnt):
        v = row_ids[pl.ds(c * CHUNK + j * 16, 16)]
        mi = jnp.minimum(v, 1)
        cs = plsc.cumsum(mi)
        pos_idx[pl.ds(j * 16, 16)] = (cnt + cs) * mi + (_PAD + _BIAS)
        return cnt + splat_last(cs)

    lax.fori_loop(0, CHUNK // 16, pos_body, cnt0)

    cp[0] = issue_p(0)
    cp[1] = issue_p(1)

    inv_hid = jnp.float32(1.0 / _HID)

    lnw_regs = [lnw_v[pl.ds(k * 16, 16)] for k in range(_HID // 16)]
    lnb_regs = [lnb_v[pl.ds(k * 16, 16)] for k in range(_HID // 16)]
    lane = lax.iota(jnp.int32, 16)
    NG = _BLK // 16

    def compute_block(wb, pb, ob):
        def tok_body(t, carry):
            es = []
            s = jnp.zeros((16,), jnp.float32)
            q = jnp.zeros((16,), jnp.float32)
            for k in range(_HID // 16):
                e = wb[t, pl.ds(k * 16, 16)] + pb[t, pl.ds(k * 16, 16)]
                es.append(e)
                s = s + e
                q = q + e * e
            mv = splat_last(plsc.cumsum(s)) * inv_hid
            ex2 = splat_last(plsc.cumsum(q)) * inv_hid
            xv = ex2 - mv * mv + _EPS
            bits = lax.bitcast_convert_type(xv, jnp.int32)
            y = lax.bitcast_convert_type(
                jnp.int32(0x5F3759DF) - lax.shift_right_logical(bits, 1),
                jnp.float32)
            for _ in range(2):
                y = y * (1.5 - 0.5 * xv * y * y)
            for k in range(_HID // 16):
                ob[t, pl.ds(k * 16, 16)] = (
                    (es[k] - mv) * y * lnw_regs[k] + lnb_regs[k])
            return carry

        lax.fori_loop(0, _BLK, tok_body, 0, unroll=8)

    for blk in range(NBLK):
        s = blk % 2
        cw[blk].wait()
        cp[blk].wait()
        if blk >= 2:
            co[blk - 2].wait()
        compute_block(wbufs[s], pbufs[s], obufs[s])
        co[blk] = pltpu.async_copy(
            obufs[s], out_hbm.at[pl.ds(wid * CHUNK + blk * _BLK, _BLK)],
            semos[s])
        if blk + 2 < NBLK:
            cw[blk + 2] = issue_w(blk + 2)
            cp[blk + 2] = issue_p(blk + 2)
    co[NBLK - 2].wait()
    co[NBLK - 1].wait()


def kernel(input_ids, word_embeddings, ln_weight, ln_bias):
    B, S = input_ids.shape
    HID = word_embeddings.shape[1]
    TOK = B * S
    CHUNK = TOK // _NW
    ROW_W = S // CHUNK

    mesh = plsc.VectorSubcoreMesh(core_axis_name="c", subcore_axis_name="s")
    run = functools.partial(
        pl.kernel,
        out_type=jax.ShapeDtypeStruct((TOK, HID), jnp.float32),
        mesh=mesh,
        compiler_params=pltpu.CompilerParams(needs_layout_passes=False),
        scratch_types=[
            pltpu.VMEM((S,), jnp.int32),
            pltpu.VMEM((CHUNK,), jnp.int32),
            pltpu.VMEM((_BLK, HID), jnp.float32),
            pltpu.VMEM((_BLK, HID), jnp.float32),
            pltpu.VMEM((_BLK, HID), jnp.float32),
            pltpu.VMEM((_BLK, HID), jnp.float32),
            pltpu.VMEM((_BLK, HID), jnp.float32),
            pltpu.VMEM((_BLK, HID), jnp.float32),
            pltpu.VMEM((HID,), jnp.float32),
            pltpu.VMEM((HID,), jnp.float32),
            pltpu.SemaphoreType.DMA,
            pltpu.SemaphoreType.DMA,
            pltpu.SemaphoreType.DMA,
            pltpu.SemaphoreType.DMA,
            pltpu.SemaphoreType.DMA,
            pltpu.SemaphoreType.DMA,
        ],
    )(functools.partial(_sc_body, S, CHUNK, ROW_W))

    pos_tab = jnp.asarray(_POS_TABLE)
    out = run(input_ids.reshape(-1), word_embeddings, pos_tab,
              ln_weight, ln_bias)
    return out.reshape(B, S, HID)


# speedup vs baseline: 1.2809x; 1.2809x over previous
import functools

import numpy as np
import jax
import jax.numpy as jnp
from jax import lax
from jax.experimental import pallas as pl
from jax.experimental.pallas import tpu as pltpu
from jax.experimental.pallas import tpu_sc as plsc

_VOCAB = 100000
_HID = 128
_PAD = 0
_MAXPOS = 16384
_BIAS = 1
_EPS = 1e-12

_NW = 32
_BLK = 128


def _sinusoidal_table(num_embeddings, embedding_dim, padding_idx):
    half_dim = embedding_dim // 2
    emb = np.log(10000.0) / (half_dim - 1)
    emb = np.exp(np.arange(half_dim, dtype=np.float64) * -emb)
    emb = np.arange(num_embeddings, dtype=np.float64)[:, None] * emb[None, :]
    table = np.concatenate([np.sin(emb), np.cos(emb)], axis=1)
    if embedding_dim % 2 == 1:
        table = np.concatenate([table, np.zeros((num_embeddings, 1))], axis=1)
    if padding_idx is not None:
        table[padding_idx, :] = 0.0
    return np.asarray(table, dtype=np.float32)


_POS_TABLE = _sinusoidal_table(_MAXPOS, _HID, _PAD)


def _sc_body(S, CHUNK, ROW_W,
             ids_hbm, wtab_hbm, ptab_hbm, lnw_hbm, lnb_hbm, out_hbm,
             row_ids, pos_idx, wbuf0, wbuf1, pbuf0, pbuf1, obuf0, obuf1,
             lnw_v, lnb_v,
             semw0, semw1, semp0, semp1, semo0, semo1):
    wid = lax.axis_index("c") * 16 + lax.axis_index("s")
    b = wid // ROW_W
    c = wid % ROW_W
    NBLK = CHUNK // _BLK

    wbufs = [wbuf0, wbuf1]
    pbufs = [pbuf0, pbuf1]
    obufs = [obuf0, obuf1]
    semws = [semw0, semw1]
    semps = [semp0, semp1]
    semos = [semo0, semo1]

    pltpu.sync_copy(ids_hbm.at[pl.ds(b * S, S)], row_ids)
    pltpu.sync_copy(lnw_hbm, lnw_v)
    pltpu.sync_copy(lnb_hbm, lnb_v)

    def issue_w(blk):
        s = blk % 2
        return pltpu.async_copy(
            wtab_hbm.at[row_ids.at[pl.ds(c * CHUNK + blk * _BLK, _BLK)]],
            wbufs[s], semws[s])

    def issue_p(blk):
        s = blk % 2
        return pltpu.async_copy(
            ptab_hbm.at[pos_idx.at[pl.ds(blk * _BLK, _BLK)]],
            pbufs[s], semps[s])

    cw = [None] * NBLK
    cp = [None] * NBLK
    co = [None] * NBLK
    cw[0] = issue_w(0)
    cw[1] = issue_w(1)

    nv = c * (CHUNK // 16)

    def pc_body(j, acc):
        v = row_ids[pl.ds(j * 16, 16)]
        return acc + jnp.minimum(v, 1)

    lane15 = jnp.full((16,), 15, jnp.int32)

    def splat_last(x):
        return x.at[lane15].get(mode="promise_in_bounds")

    acc = lax.fori_loop(0, nv, pc_body, jnp.zeros((16,), jnp.int32))
    cnt0 = splat_last(plsc.cumsum(acc))

    def pos_body(j, cnt):
        v = row_ids[pl.ds(c * CHUNK + j * 16, 16)]
        mi = jnp.minimum(v, 1)
        cs = plsc.cumsum(mi)
        pos_idx[pl.ds(j * 16, 16)] = (cnt + cs) * mi + (_PAD + _BIAS)
        return cnt + splat_last(cs)

    lax.fori_loop(0, CHUNK // 16, pos_body, cnt0)

    cp[0] = issue_p(0)
    cp[1] = issue_p(1)

    inv_hid = jnp.float32(1.0 / _HID)

    lnw_regs = [lnw_v[pl.ds(k * 16, 16)] for k in range(_HID // 16)]
    lnb_regs = [lnb_v[pl.ds(k * 16, 16)] for k in range(_HID // 16)]
    lane = lax.iota(jnp.int32, 16)
    NG = _BLK // 16

    def compute_block(wb, pb, ob):
        def load_stats(t):
            es = []
            s = jnp.zeros((16,), jnp.float32)
            q = jnp.zeros((16,), jnp.float32)
            for k in range(_HID // 16):
                e = wb[t, pl.ds(k * 16, 16)] + pb[t, pl.ds(k * 16, 16)]
                es.append(e)
                s = s + e
                q = q + e * e
            return es, s, q

        def newton(s, q):
            mv = splat_last(plsc.cumsum(s)) * inv_hid
            ex2 = splat_last(plsc.cumsum(q)) * inv_hid
            xv = ex2 - mv * mv + _EPS
            bits = lax.bitcast_convert_type(xv, jnp.int32)
            y = lax.bitcast_convert_type(
                jnp.int32(0x5F3759DF) - lax.shift_right_logical(bits, 1),
                jnp.float32)
            for _ in range(2):
                y = y * (1.5 - 0.5 * xv * y * y)
            return mv, y

        def norm_store(t, es, mv, y):
            for k in range(_HID // 16):
                ob[t, pl.ds(k * 16, 16)] = (
                    (es[k] - mv) * y * lnw_regs[k] + lnb_regs[k])

        def pair_body(i, carry):
            t0 = 2 * i
            t1 = 2 * i + 1
            es0, s0, q0 = load_stats(t0)
            es1, s1, q1 = load_stats(t1)
            mv0, y0 = newton(s0, q0)
            mv1, y1 = newton(s1, q1)
            norm_store(t0, es0, mv0, y0)
            norm_store(t1, es1, mv1, y1)
            return carry

        lax.fori_loop(0, _BLK // 2, pair_body, 0, unroll=2)

    for blk in range(NBLK):
        s = blk % 2
        cw[blk].wait()
        cp[blk].wait()
        if blk >= 2:
            co[blk - 2].wait()
        compute_block(wbufs[s], pbufs[s], obufs[s])
        co[blk] = pltpu.async_copy(
            obufs[s], out_hbm.at[pl.ds(wid * CHUNK + blk * _BLK, _BLK)],
            semos[s])
        if blk + 2 < NBLK:
            cw[blk + 2] = issue_w(blk + 2)
            cp[blk + 2] = issue_p(blk + 2)
    co[NBLK - 2].wait()
    co[NBLK - 1].wait()


def kernel(input_ids, word_embeddings, ln_weight, ln_bias):
    B, S = input_ids.shape
    HID = word_embeddings.shape[1]
    TOK = B * S
    CHUNK = TOK // _NW
    ROW_W = S // CHUNK

    mesh = plsc.VectorSubcoreMesh(core_axis_name="c", subcore_axis_name="s")
    run = functools.partial(
        pl.kernel,
        out_type=jax.ShapeDtypeStruct((TOK, HID), jnp.float32),
        mesh=mesh,
        compiler_params=pltpu.CompilerParams(needs_layout_passes=False),
        scratch_types=[
            pltpu.VMEM((S,), jnp.int32),
            pltpu.VMEM((CHUNK,), jnp.int32),
            pltpu.VMEM((_BLK, HID), jnp.float32),
            pltpu.VMEM((_BLK, HID), jnp.float32),
            pltpu.VMEM((_BLK, HID), jnp.float32),
            pltpu.VMEM((_BLK, HID), jnp.float32),
            pltpu.VMEM((_BLK, HID), jnp.float32),
            pltpu.VMEM((_BLK, HID), jnp.float32),
            pltpu.VMEM((HID,), jnp.float32),
            pltpu.VMEM((HID,), jnp.float32),
            pltpu.SemaphoreType.DMA,
            pltpu.SemaphoreType.DMA,
            pltpu.SemaphoreType.DMA,
            pltpu.SemaphoreType.DMA,
            pltpu.SemaphoreType.DMA,
            pltpu.SemaphoreType.DMA,
        ],
    )(functools.partial(_sc_body, S, CHUNK, ROW_W))

    pos_tab = jnp.asarray(_POS_TABLE)
    out = run(input_ids.reshape(-1), word_embeddings, pos_tab,
              ln_weight, ln_bias)
    return out.reshape(B, S, HID)


# speedup vs baseline: 1.3434x; 1.0488x over previous
import functools

import numpy as np
import jax
import jax.numpy as jnp
from jax import lax
from jax.experimental import pallas as pl
from jax.experimental.pallas import tpu as pltpu
from jax.experimental.pallas import tpu_sc as plsc

_VOCAB = 100000
_HID = 128
_PAD = 0
_MAXPOS = 16384
_BIAS = 1
_EPS = 1e-12

_NW = 32
_BLK = 128


def _sinusoidal_table(num_embeddings, embedding_dim, padding_idx):
    half_dim = embedding_dim // 2
    emb = np.log(10000.0) / (half_dim - 1)
    emb = np.exp(np.arange(half_dim, dtype=np.float64) * -emb)
    emb = np.arange(num_embeddings, dtype=np.float64)[:, None] * emb[None, :]
    table = np.concatenate([np.sin(emb), np.cos(emb)], axis=1)
    if embedding_dim % 2 == 1:
        table = np.concatenate([table, np.zeros((num_embeddings, 1))], axis=1)
    if padding_idx is not None:
        table[padding_idx, :] = 0.0
    return np.asarray(table, dtype=np.float32)


_POS_TABLE = _sinusoidal_table(_MAXPOS, _HID, _PAD)


def _sc_body(S, CHUNK, ROW_W,
             ids_hbm, wtab_hbm, ptab_hbm, lnw_hbm, lnb_hbm, out_hbm,
             row_ids, pos_idx, wbuf0, wbuf1, pbuf0, pbuf1, obuf0, obuf1,
             lnw_v, lnb_v,
             semw0, semw1, semp0, semp1, semo0, semo1):
    wid = lax.axis_index("c") * 16 + lax.axis_index("s")
    b = wid // ROW_W
    c = wid % ROW_W
    NBLK = CHUNK // _BLK

    wbufs = [wbuf0, wbuf1]
    pbufs = [pbuf0, pbuf1]
    obufs = [obuf0, obuf1]
    semws = [semw0, semw1]
    semps = [semp0, semp1]
    semos = [semo0, semo1]

    pltpu.sync_copy(ids_hbm.at[pl.ds(b * S, S)], row_ids)
    pltpu.sync_copy(lnw_hbm, lnw_v)
    pltpu.sync_copy(lnb_hbm, lnb_v)

    def issue_w(blk):
        s = blk % 2
        return pltpu.async_copy(
            wtab_hbm.at[row_ids.at[pl.ds(c * CHUNK + blk * _BLK, _BLK)]],
            wbufs[s], semws[s])

    def issue_p(blk):
        s = blk % 2
        return pltpu.async_copy(
            ptab_hbm.at[pos_idx.at[pl.ds(blk * _BLK, _BLK)]],
            pbufs[s], semps[s])

    cw = [None] * NBLK
    cp = [None] * NBLK
    co = [None] * NBLK
    cw[0] = issue_w(0)
    cw[1] = issue_w(1)

    nv = c * (CHUNK // 16)

    def pc_body(j, acc):
        v = row_ids[pl.ds(j * 16, 16)]
        return acc + jnp.minimum(v, 1)

    lane15 = jnp.full((16,), 15, jnp.int32)

    def splat_last(x):
        return x.at[lane15].get(mode="promise_in_bounds")

    acc = lax.fori_loop(0, nv, pc_body, jnp.zeros((16,), jnp.int32))
    cnt0 = splat_last(plsc.cumsum(acc))

    def pos_body(j, cnt):
        v = row_ids[pl.ds(c * CHUNK + j * 16, 16)]
        mi = jnp.minimum(v, 1)
        cs = plsc.cumsum(mi)
        pos_idx[pl.ds(j * 16, 16)] = (cnt + cs) * mi + (_PAD + _BIAS)
        return cnt + splat_last(cs)

    lax.fori_loop(0, CHUNK // 16, pos_body, cnt0)

    cp[0] = issue_p(0)
    cp[1] = issue_p(1)

    inv_hid = jnp.float32(1.0 / _HID)

    lnw_regs = [lnw_v[pl.ds(k * 16, 16)] for k in range(_HID // 16)]
    lnb_regs = [lnb_v[pl.ds(k * 16, 16)] for k in range(_HID // 16)]
    lane = lax.iota(jnp.int32, 16)
    NG = _BLK // 16

    def compute_block(wb, pb, ob):
        def load_stats(t):
            es = []
            s = jnp.zeros((16,), jnp.float32)
            q = jnp.zeros((16,), jnp.float32)
            for k in range(_HID // 16):
                e = wb[t, pl.ds(k * 16, 16)] + pb[t, pl.ds(k * 16, 16)]
                es.append(e)
                s = s + e
                q = q + e * e
            return es, s, q

        def newton(s, q):
            mv = splat_last(plsc.cumsum(s)) * inv_hid
            ex2 = splat_last(plsc.cumsum(q)) * inv_hid
            xv = ex2 - mv * mv + _EPS
            bits = lax.bitcast_convert_type(xv, jnp.int32)
            y = lax.bitcast_convert_type(
                jnp.int32(0x5F3759DF) - lax.shift_right_logical(bits, 1),
                jnp.float32)
            for _ in range(2):
                y = y * (1.5 - 0.5 * xv * y * y)
            return mv, y

        def norm_store(t, es, mv, y):
            for k in range(_HID // 16):
                ob[t, pl.ds(k * 16, 16)] = (
                    (es[k] - mv) * y * lnw_regs[k] + lnb_regs[k])

        def quad_body(i, carry):
            ts = [4 * i + d for d in range(4)]
            sq = [load_stats(t) for t in ts]
            st = [newton(s, q) for (_, s, q) in sq]
            for t, (es, _, _), (mv, y) in zip(ts, sq, st):
                norm_store(t, es, mv, y)
            return carry

        lax.fori_loop(0, _BLK // 4, quad_body, 0)

    for blk in range(NBLK):
        s = blk % 2
        cw[blk].wait()
        cp[blk].wait()
        if blk >= 2:
            co[blk - 2].wait()
        compute_block(wbufs[s], pbufs[s], obufs[s])
        co[blk] = pltpu.async_copy(
            obufs[s], out_hbm.at[pl.ds(wid * CHUNK + blk * _BLK, _BLK)],
            semos[s])
        if blk + 2 < NBLK:
            cw[blk + 2] = issue_w(blk + 2)
            cp[blk + 2] = issue_p(blk + 2)
    co[NBLK - 2].wait()
    co[NBLK - 1].wait()


def kernel(input_ids, word_embeddings, ln_weight, ln_bias):
    B, S = input_ids.shape
    HID = word_embeddings.shape[1]
    TOK = B * S
    CHUNK = TOK // _NW
    ROW_W = S // CHUNK

    mesh = plsc.VectorSubcoreMesh(core_axis_name="c", subcore_axis_name="s")
    run = functools.partial(
        pl.kernel,
        out_type=jax.ShapeDtypeStruct((TOK, HID), jnp.float32),
        mesh=mesh,
        compiler_params=pltpu.CompilerParams(needs_layout_passes=False),
        scratch_types=[
            pltpu.VMEM((S,), jnp.int32),
            pltpu.VMEM((CHUNK,), jnp.int32),
            pltpu.VMEM((_BLK, HID), jnp.float32),
            pltpu.VMEM((_BLK, HID), jnp.float32),
            pltpu.VMEM((_BLK, HID), jnp.float32),
            pltpu.VMEM((_BLK, HID), jnp.float32),
            pltpu.VMEM((_BLK, HID), jnp.float32),
            pltpu.VMEM((_BLK, HID), jnp.float32),
            pltpu.VMEM((HID,), jnp.float32),
            pltpu.VMEM((HID,), jnp.float32),
            pltpu.SemaphoreType.DMA,
            pltpu.SemaphoreType.DMA,
            pltpu.SemaphoreType.DMA,
            pltpu.SemaphoreType.DMA,
            pltpu.SemaphoreType.DMA,
            pltpu.SemaphoreType.DMA,
        ],
    )(functools.partial(_sc_body, S, CHUNK, ROW_W))

    pos_tab = jnp.asarray(_POS_TABLE)
    out = run(input_ids.reshape(-1), word_embeddings, pos_tab,
              ln_weight, ln_bias)
    return out.reshape(B, S, HID)


# speedup vs baseline: 1.4474x; 1.0774x over previous
import functools

import numpy as np
import jax
import jax.numpy as jnp
from jax import lax
from jax.experimental import pallas as pl
from jax.experimental.pallas import tpu as pltpu
from jax.experimental.pallas import tpu_sc as plsc

_VOCAB = 100000
_HID = 128
_PAD = 0
_MAXPOS = 16384
_BIAS = 1
_EPS = 1e-12

_NW = 32
_BLK = 128


def _sinusoidal_table(num_embeddings, embedding_dim, padding_idx):
    half_dim = embedding_dim // 2
    emb = np.log(10000.0) / (half_dim - 1)
    emb = np.exp(np.arange(half_dim, dtype=np.float64) * -emb)
    emb = np.arange(num_embeddings, dtype=np.float64)[:, None] * emb[None, :]
    table = np.concatenate([np.sin(emb), np.cos(emb)], axis=1)
    if embedding_dim % 2 == 1:
        table = np.concatenate([table, np.zeros((num_embeddings, 1))], axis=1)
    if padding_idx is not None:
        table[padding_idx, :] = 0.0
    return np.asarray(table, dtype=np.float32)


_POS_TABLE = _sinusoidal_table(_MAXPOS, _HID, _PAD)


def _sc_body(S, CHUNK, ROW_W,
             ids_hbm, wtab_hbm, ptab_hbm, lnw_hbm, lnb_hbm, out_hbm,
             row_ids, pos_idx, wbuf0, wbuf1, pbuf0, pbuf1, obuf0, obuf1,
             lnw_v, lnb_v,
             semw0, semw1, semp0, semp1, semo0, semo1):
    wid = lax.axis_index("c") * 16 + lax.axis_index("s")
    b = wid // ROW_W
    c = wid % ROW_W
    NBLK = CHUNK // _BLK

    wbufs = [wbuf0, wbuf1]
    pbufs = [pbuf0, pbuf1]
    obufs = [obuf0, obuf1]
    semws = [semw0, semw1]
    semps = [semp0, semp1]
    semos = [semo0, semo1]

    pltpu.sync_copy(ids_hbm.at[pl.ds(b * S, S)], row_ids)
    pltpu.sync_copy(lnw_hbm, lnw_v)
    pltpu.sync_copy(lnb_hbm, lnb_v)

    def issue_w(blk):
        s = blk % 2
        return pltpu.async_copy(
            wtab_hbm.at[row_ids.at[pl.ds(c * CHUNK + blk * _BLK, _BLK)]],
            wbufs[s], semws[s])

    def issue_p(blk):
        s = blk % 2
        return pltpu.async_copy(
            ptab_hbm.at[pos_idx.at[pl.ds(blk * _BLK, _BLK)]],
            pbufs[s], semps[s])

    cw = [None] * NBLK
    cp = [None] * NBLK
    co = [None] * NBLK
    cw[0] = issue_w(0)
    cw[1] = issue_w(1)

    nv = c * (CHUNK // 16)

    def pc_body(j, acc):
        v = row_ids[pl.ds(j * 16, 16)]
        m = jnp.minimum(jnp.maximum(nv - j, 0), 1)
        return acc + m * jnp.minimum(v, 1)

    lane15 = jnp.full((16,), 15, jnp.int32)

    def splat_last(x):
        return x.at[lane15].get(mode="promise_in_bounds")

    acc = lax.fori_loop(0, (ROW_W - 1) * (CHUNK // 16), pc_body,
                        jnp.zeros((16,), jnp.int32), unroll=8)
    cnt0 = splat_last(plsc.cumsum(acc))

    def pos_body(j, cnt):
        v = row_ids[pl.ds(c * CHUNK + j * 16, 16)]
        mi = jnp.minimum(v, 1)
        cs = plsc.cumsum(mi)
        pos_idx[pl.ds(j * 16, 16)] = (cnt + cs) * mi + (_PAD + _BIAS)
        return cnt + splat_last(cs)

    lax.fori_loop(0, CHUNK // 16, pos_body, cnt0)

    cp[0] = issue_p(0)
    cp[1] = issue_p(1)

    inv_hid = jnp.float32(1.0 / _HID)

    lnw_regs = [lnw_v[pl.ds(k * 16, 16)] for k in range(_HID // 16)]
    lnb_regs = [lnb_v[pl.ds(k * 16, 16)] for k in range(_HID // 16)]
    lane = lax.iota(jnp.int32, 16)
    NG = _BLK // 16

    def compute_block(wb, pb, ob):
        def load_stats(t):
            es = []
            s = jnp.zeros((16,), jnp.float32)
            q = jnp.zeros((16,), jnp.float32)
            for k in range(_HID // 16):
                e = wb[t, pl.ds(k * 16, 16)] + pb[t, pl.ds(k * 16, 16)]
                es.append(e)
                s = s + e
                q = q + e * e
            return es, s, q

        def newton(s, q):
            mv = splat_last(plsc.cumsum(s)) * inv_hid
            ex2 = splat_last(plsc.cumsum(q)) * inv_hid
            xv = ex2 - mv * mv + _EPS
            bits = lax.bitcast_convert_type(xv, jnp.int32)
            y = lax.bitcast_convert_type(
                jnp.int32(0x5F3759DF) - lax.shift_right_logical(bits, 1),
                jnp.float32)
            for _ in range(2):
                y = y * (1.5 - 0.5 * xv * y * y)
            return mv, y

        def norm_store(t, es, mv, y):
            for k in range(_HID // 16):
                ob[t, pl.ds(k * 16, 16)] = (
                    (es[k] - mv) * y * lnw_regs[k] + lnb_regs[k])

        def quad_body(i, carry):
            ts = [4 * i + d for d in range(4)]
            sq = [load_stats(t) for t in ts]
            st = [newton(s, q) for (_, s, q) in sq]
            for t, (es, _, _), (mv, y) in zip(ts, sq, st):
                norm_store(t, es, mv, y)
            return carry

        lax.fori_loop(0, _BLK // 4, quad_body, 0, unroll=2)

    for blk in range(NBLK):
        s = blk % 2
        cw[blk].wait()
        cp[blk].wait()
        if blk >= 2:
            co[blk - 2].wait()
        compute_block(wbufs[s], pbufs[s], obufs[s])
        co[blk] = pltpu.async_copy(
            obufs[s], out_hbm.at[pl.ds(wid * CHUNK + blk * _BLK, _BLK)],
            semos[s])
        if blk + 2 < NBLK:
            cw[blk + 2] = issue_w(blk + 2)
            cp[blk + 2] = issue_p(blk + 2)
    co[NBLK - 2].wait()
    co[NBLK - 1].wait()


def kernel(input_ids, word_embeddings, ln_weight, ln_bias):
    B, S = input_ids.shape
    HID = word_embeddings.shape[1]
    TOK = B * S
    CHUNK = TOK // _NW
    ROW_W = S // CHUNK

    mesh = plsc.VectorSubcoreMesh(core_axis_name="c", subcore_axis_name="s")
    run = functools.partial(
        pl.kernel,
        out_type=jax.ShapeDtypeStruct((TOK, HID), jnp.float32),
        mesh=mesh,
        compiler_params=pltpu.CompilerParams(needs_layout_passes=False),
        scratch_types=[
            pltpu.VMEM((S,), jnp.int32),
            pltpu.VMEM((CHUNK,), jnp.int32),
            pltpu.VMEM((_BLK, HID), jnp.float32),
            pltpu.VMEM((_BLK, HID), jnp.float32),
            pltpu.VMEM((_BLK, HID), jnp.float32),
            pltpu.VMEM((_BLK, HID), jnp.float32),
            pltpu.VMEM((_BLK, HID), jnp.float32),
            pltpu.VMEM((_BLK, HID), jnp.float32),
            pltpu.VMEM((HID,), jnp.float32),
            pltpu.VMEM((HID,), jnp.float32),
            pltpu.SemaphoreType.DMA,
            pltpu.SemaphoreType.DMA,
            pltpu.SemaphoreType.DMA,
            pltpu.SemaphoreType.DMA,
            pltpu.SemaphoreType.DMA,
            pltpu.SemaphoreType.DMA,
        ],
    )(functools.partial(_sc_body, S, CHUNK, ROW_W))

    pos_tab = jnp.asarray(_POS_TABLE)
    out = run(input_ids.reshape(-1), word_embeddings, pos_tab,
              ln_weight, ln_bias)
    return out.reshape(B, S, HID)
